# baseline (speedup 1.0000x reference)
import functools

import jax
import jax.numpy as jnp
from jax import lax
from jax.experimental import pallas as pl
from jax.experimental.pallas import tpu as pltpu
from jax.experimental.pallas import tpu_sc as plsc

N = 10000
E = 320000
D = 128
H = 128
L = 4
B = 64
OUT = 128
NODE_INS = 1.0
NODE_DEL = 1.0
EDGE_INS = 1.0
EDGE_DEL = 1.0

NTILE = 16
NCORE = 2
HHALF = H // NCORE
ROWS_PT = N // NTILE
ECHUNK = 128
NBUF = 5
NCHUNK = 160
EPT = NCHUNK * ECHUNK
EPAD = NTILE * EPT - E

RBLK = 1000
NRB = N // RBLK


def _sc_body(x_hbm, qsrc, qdst, csrc, cdst, h_out,
             spx, spagg, idx_s, idx_d, rows, semis, semid, semg, sems_s):
    c = lax.axis_index("c")
    s = lax.axis_index("s")
    col = c * HHALF
    row0 = s * ROWS_PT

    for g, (src, dst) in enumerate(((qsrc, qdst), (csrc, cdst))):
        xsl = x_hbm.at[g, pl.ds(row0, ROWS_PT), pl.ds(col, HHALF)]
        pltpu.sync_copy(xsl, spx.at[pl.ds(row0, ROWS_PT), :])
        pltpu.sync_copy(xsl, spagg.at[pl.ds(row0, ROWS_PT), :])
        plsc.subcore_barrier()

        def fire_idx(b, j0):
            for i in range(NBUF):
                pltpu.async_copy(src.at[s, j0 + i], idx_s.at[b, i],
                                 semis.at[b, i])
                pltpu.async_copy(dst.at[s, j0 + i], idx_d.at[b, i],
                                 semid.at[b, i])

        def drain_scatter(i):
            pltpu.make_async_copy(
                x_hbm.at[g, pl.ds(0, ECHUNK), pl.ds(0, HHALF)],
                rows.at[i], sems_s.at[i]).wait()

        NB = NCHUNK // (2 * NBUF)

        fire_idx(0, 0)

        def half(b):
            gh = []
            for i in range(NBUF):
                pltpu.make_async_copy(src.at[s, 0], idx_s.at[b, i],
                                      semis.at[b, i]).wait()
                gh.append(pltpu.async_copy(spx.at[idx_s.at[b, i]],
                                           rows.at[i], semg.at[i]))
            return gh

        def scatters(b, gh):
            for i in range(NBUF):
                gh[i].wait()
                pltpu.make_async_copy(dst.at[s, 0], idx_d.at[b, i],
                                      semid.at[b, i]).wait()
                pltpu.sync_copy(rows.at[i], spagg.at[idx_d.at[b, i]],
                                add=True)

        def body(k, _):
            j0 = k * 2 * NBUF
            gh = half(0)
            fire_idx(1, j0 + NBUF)
            scatters(0, gh)
            gh = half(1)

            @pl.when(k < NB - 1)
            def _():
                fire_idx(0, j0 + 2 * NBUF)
            scatters(1, gh)
            return 0

        lax.fori_loop(0, NB, body, 0)

        plsc.subcore_barrier()
        pltpu.sync_copy(spagg.at[pl.ds(row0, ROWS_PT), :],
                        h_out.at[g, pl.ds(row0, ROWS_PT), pl.ds(col, HHALF)])
        plsc.subcore_barrier()


@functools.cache
def _sc_aggregate_kernel():
    return pl.kernel(
        _sc_body,
        out_type=jax.ShapeDtypeStruct((2, N, H), jnp.float32),
        mesh=plsc.VectorSubcoreMesh(core_axis_name="c", subcore_axis_name="s"),
        scratch_types=[
            pltpu.VMEM_SHARED((N, HHALF), jnp.float32),
            pltpu.VMEM_SHARED((N + 8, HHALF), jnp.float32),
            pltpu.VMEM((2, NBUF, ECHUNK), jnp.int32),
            pltpu.VMEM((2, NBUF, ECHUNK), jnp.int32),
            pltpu.VMEM((NBUF, ECHUNK, HHALF), jnp.float32),
            pltpu.SemaphoreType.DMA((2, NBUF)),
            pltpu.SemaphoreType.DMA((2, NBUF)),
            pltpu.SemaphoreType.DMA((NBUF,)),
            pltpu.SemaphoreType.DMA((NBUF,)),
        ],
        compiler_params=pltpu.CompilerParams(use_tc_tiling_on_sc=False),
    )


def _pool_accum(pooled_ref, batch_ref, xnew):
    r = pl.program_id(1)
    ids = batch_ref[0, 0, :]
    seg = lax.broadcasted_iota(jnp.int32, (B, RBLK), 0)
    onehot = (seg == ids[None, :]).astype(jnp.float32)
    part = jnp.dot(onehot, xnew, preferred_element_type=jnp.float32)

    @pl.when(r == 0)
    def _():
        pooled_ref[0] = part

    @pl.when(r != 0)
    def _():
        pooled_ref[0] += part


def _pre_body(x_ref, w_ref, b_ref, batch_ref, x0_ref, pooled_ref):
    x0 = jnp.dot(x_ref[0], w_ref[...], preferred_element_type=jnp.float32)
    x0 = x0 + b_ref[0]
    x0_ref[0] = x0
    _pool_accum(pooled_ref, batch_ref, x0)


def _layer_body(odd, h_ref, w1_ref, b1_ref, w2_ref, b2_ref, batch_ref,
                *rest):
    if odd:
        xres_ref, x_ref, pooled_ref, xres_out_ref = rest
    else:
        x_ref, pooled_ref = rest
    a = jnp.dot(h_ref[0], w1_ref[...], preferred_element_type=jnp.float32)
    a = jnp.maximum(a + b1_ref[0], 0.0)
    x2 = jnp.dot(a, w2_ref[...], preferred_element_type=jnp.float32)
    x2 = x2 + b2_ref[0]
    if odd:
        x2 = x2 + xres_ref[0]
        xres_out_ref[0] = x2
    xn = jnp.maximum(x2, 0.0)
    x_ref[0] = xn
    _pool_accum(pooled_ref, batch_ref, xn)


def _final_body(pq_ref, pc_ref, w1_ref, b1_ref, w2_ref, b2_ref, out_ref):
    def head(p):
        a = jnp.dot(p, w1_ref[...], preferred_element_type=jnp.float32)
        a = jnp.maximum(a + b1_ref[0], 0.0)
        return jnp.dot(a, w2_ref[...],
                       preferred_element_type=jnp.float32) + b2_ref[0]

    gx = head(pq_ref[...])
    hx = head(pc_ref[...])
    d = gx - hx
    a = jnp.sum(jnp.maximum(d, 0.0), axis=-1) * ((NODE_DEL + EDGE_DEL) * 0.5)
    b = jnp.sum(jnp.maximum(-d, 0.0), axis=-1) * ((NODE_INS + EDGE_INS) * 0.5)
    out_ref[0] = a + b


def _row_specs(n_extra_like_x):
    xspec = pl.BlockSpec((1, RBLK, H), lambda g, r: (g, r, 0))
    return xspec


_XSPEC = pl.BlockSpec((1, RBLK, H), lambda g, r: (g, r, 0))
_WSPEC = pl.BlockSpec((H, H), lambda g, r: (0, 0))
_BSPEC = pl.BlockSpec((1, H), lambda g, r: (0, 0))
_BATCHSPEC = pl.BlockSpec((1, 1, RBLK), lambda g, r: (g * NRB + r, 0, 0))
_POOLSPEC = pl.BlockSpec((1, B, H), lambda g, r: (g, 0, 0))


def _pre_call(x, Wpre, bpre, batch3):
    return pl.pallas_call(
        _pre_body,
        grid=(2, NRB),
        in_specs=[_XSPEC, _WSPEC, _BSPEC, _BATCHSPEC],
        out_specs=[_XSPEC, _POOLSPEC],
        out_shape=[jax.ShapeDtypeStruct((2, N, H), jnp.float32),
                   jax.ShapeDtypeStruct((2, B, H), jnp.float32)],
    )(x, Wpre, bpre, batch3)


def _layer_call(odd, h, W1, b1, W2, b2, batch3, xres):
    in_specs = [_XSPEC, _WSPEC, _BSPEC, _WSPEC, _BSPEC, _BATCHSPEC]
    operands = [h, W1, b1, W2, b2, batch3]
    out_specs = [_XSPEC, _POOLSPEC]
    out_shape = [jax.ShapeDtypeStruct((2, N, H), jnp.float32),
                 jax.ShapeDtypeStruct((2, B, H), jnp.float32)]
    if odd:
        in_specs.append(_XSPEC)
        operands.append(xres)
        out_specs.append(_XSPEC)
        out_shape.append(jax.ShapeDtypeStruct((2, N, H), jnp.float32))
    return pl.pallas_call(
        functools.partial(_layer_body, odd),
        grid=(2, NRB),
        in_specs=in_specs,
        out_specs=out_specs,
        out_shape=out_shape,
    )(*operands)


def _final_call(pq, pc, Wp1, bp1, Wp2, bp2):
    full = lambda shape: pl.BlockSpec(shape, lambda: tuple(0 for _ in shape))
    return pl.pallas_call(
        _final_body,
        in_specs=[full((B, H * (L + 1))), full((B, H * (L + 1))),
                  full((H * (L + 1), H)), full((1, H)),
                  full((H, OUT)), full((1, OUT))],
        out_specs=full((1, B)),
        out_shape=jax.ShapeDtypeStruct((1, B), jnp.float32),
    )(pq, pc, Wp1, bp1, Wp2, bp2)


def _sc_h_all(x, qsrc, qdst, csrc, cdst):
    return _sc_aggregate_kernel()(x, qsrc, qdst, csrc, cdst)


def kernel(query_x, query_edge_index, query_batch, corpus_x,
           corpus_edge_index, corpus_batch, Wpre, bpre, Wc1, bc1, Wc2, bc2,
           Wp1, bp1, Wp2, bp2):
    x_in = jnp.stack([query_x, corpus_x])
    batch3 = jnp.stack([query_batch, corpus_batch]).reshape(2 * NRB, 1, RBLK)

    pad_src = jnp.arange(EPAD, dtype=jnp.int32) % 256
    pad_dst = N + jnp.arange(EPAD, dtype=jnp.int32) % 8

    def prep_edges(ei):
        src = jnp.concatenate([ei[0], pad_src])
        dst = jnp.concatenate([ei[1], pad_dst])
        return (src.reshape(NTILE, NCHUNK, ECHUNK),
                dst.reshape(NTILE, NCHUNK, ECHUNK))

    qsrc, qdst = prep_edges(query_edge_index)
    csrc, cdst = prep_edges(corpus_edge_index)
    bpre2 = bpre.reshape(1, H)

    x, p0 = _pre_call(x_in, Wpre, bpre2, batch3)
    pooled = [p0]
    xres = x
    for i in range(L):
        h = _sc_h_all(x, qsrc, qdst, csrc, cdst)
        outs = _layer_call(i & 1, h, Wc1[i], bc1[i].reshape(1, H),
                           Wc2[i], bc2[i].reshape(1, H), batch3, xres)
        if i & 1:
            x, pi, xres = outs
        else:
            x, pi = outs
        pooled.append(pi)

    pall = jnp.concatenate(pooled, axis=-1)
    score = _final_call(pall[0], pall[1], Wp1, bp1.reshape(1, H),
                        Wp2, bp2.reshape(1, OUT))
    return score.reshape(B)


# speedup vs baseline: 1.4601x; 1.4601x over previous
import functools

import jax
import jax.numpy as jnp
from jax import lax
from jax.experimental import pallas as pl
from jax.experimental.pallas import tpu as pltpu
from jax.experimental.pallas import tpu_sc as plsc

N = 10000
E = 320000
D = 128
H = 128
L = 4
B = 64
OUT = 128
NODE_INS = 1.0
NODE_DEL = 1.0
EDGE_INS = 1.0
EDGE_DEL = 1.0

NTILE = 16
NCORE = 2
HHALF = H // NCORE
ROWS_PT = N // NTILE
ECHUNK = 128
NBUF = 4
NCHUNK = 160
EPT = NCHUNK * ECHUNK
EPAD = NTILE * EPT - E

RBLK = 1000
NRB = N // RBLK


def _sc_body(x_hbm, src, dst, h_out,
             spx, spagg, idx_s, idx_d, rows, semis, semid, semg):
    c = lax.axis_index("c")
    s = lax.axis_index("s")
    col = c * HHALF
    row0 = s * ROWS_PT

    xsl = x_hbm.at[pl.ds(row0, ROWS_PT), pl.ds(col, HHALF)]
    pltpu.sync_copy(xsl, spx.at[pl.ds(row0, ROWS_PT), :])
    pltpu.sync_copy(xsl, spagg.at[pl.ds(row0, ROWS_PT), :])
    plsc.subcore_barrier()

    def fire_idx(b, j0):
        for i in range(NBUF):
            pltpu.async_copy(src.at[s, j0 + i], idx_s.at[b, i],
                             semis.at[b, i])
            pltpu.async_copy(dst.at[s, j0 + i], idx_d.at[b, i],
                             semid.at[b, i])

    NB = NCHUNK // (2 * NBUF)

    fire_idx(0, 0)

    def half(b):
        gh = []
        for i in range(NBUF):
            pltpu.make_async_copy(src.at[s, 0], idx_s.at[b, i],
                                  semis.at[b, i]).wait()
            gh.append(pltpu.async_copy(spx.at[idx_s.at[b, i]],
                                       rows.at[i], semg.at[i]))
        return gh

    def scatters(b, gh):
        for i in range(NBUF):
            gh[i].wait()
            pltpu.make_async_copy(dst.at[s, 0], idx_d.at[b, i],
                                  semid.at[b, i]).wait()
            pltpu.sync_copy(rows.at[i], spagg.at[idx_d.at[b, i]],
                            add=True)

    def body(k, _):
        j0 = k * 2 * NBUF
        gh = half(0)
        fire_idx(1, j0 + NBUF)
        scatters(0, gh)
        gh = half(1)

        @pl.when(k < NB - 1)
        def _():
            fire_idx(0, j0 + 2 * NBUF)
        scatters(1, gh)
        return 0

    lax.fori_loop(0, NB, body, 0)

    plsc.subcore_barrier()
    pltpu.sync_copy(spagg.at[pl.ds(row0, ROWS_PT), :],
                    h_out.at[pl.ds(row0, ROWS_PT), pl.ds(col, HHALF)])
    plsc.subcore_barrier()


@functools.cache
def _sc_aggregate_kernel():
    return pl.kernel(
        _sc_body,
        out_type=jax.ShapeDtypeStruct((N, H), jnp.float32),
        mesh=plsc.VectorSubcoreMesh(core_axis_name="c", subcore_axis_name="s"),
        scratch_types=[
            pltpu.VMEM_SHARED((N, HHALF), jnp.float32),
            pltpu.VMEM_SHARED((N + 8, HHALF), jnp.float32),
            pltpu.VMEM((2, NBUF, ECHUNK), jnp.int32),
            pltpu.VMEM((2, NBUF, ECHUNK), jnp.int32),
            pltpu.VMEM((NBUF, ECHUNK, HHALF), jnp.float32),
            pltpu.SemaphoreType.DMA((2, NBUF)),
            pltpu.SemaphoreType.DMA((2, NBUF)),
            pltpu.SemaphoreType.DMA((NBUF,)),
        ],
        compiler_params=pltpu.CompilerParams(use_tc_tiling_on_sc=False),
    )


def _pool_accum(pooled_ref, batch_ref, xnew):
    r = pl.program_id(0)
    ids = batch_ref[0, 0, :]
    seg = lax.broadcasted_iota(jnp.int32, (B, RBLK), 0)
    onehot = (seg == ids[None, :]).astype(jnp.float32)
    part = jnp.dot(onehot, xnew, preferred_element_type=jnp.float32)

    @pl.when(r == 0)
    def _():
        pooled_ref[...] = part

    @pl.when(r != 0)
    def _():
        pooled_ref[...] += part


def _pre_body(x_ref, w_ref, b_ref, batch_ref, x0_ref, pooled_ref):
    x0 = jnp.dot(x_ref[...], w_ref[...], preferred_element_type=jnp.float32)
    x0 = x0 + b_ref[0]
    x0_ref[...] = x0
    _pool_accum(pooled_ref, batch_ref, x0)


def _layer_body(odd, h_ref, w1_ref, b1_ref, w2_ref, b2_ref, batch_ref,
                *rest):
    if odd:
        xres_ref, x_ref, pooled_ref, xres_out_ref = rest
    else:
        x_ref, pooled_ref = rest
    a = jnp.dot(h_ref[...], w1_ref[...], preferred_element_type=jnp.float32)
    a = jnp.maximum(a + b1_ref[0], 0.0)
    x2 = jnp.dot(a, w2_ref[...], preferred_element_type=jnp.float32)
    x2 = x2 + b2_ref[0]
    if odd:
        x2 = x2 + xres_ref[...]
        xres_out_ref[...] = x2
    xn = jnp.maximum(x2, 0.0)
    x_ref[...] = xn
    _pool_accum(pooled_ref, batch_ref, xn)


def _final_body(pq_ref, pc_ref, w1_ref, b1_ref, w2_ref, b2_ref, out_ref):
    def head(p):
        a = jnp.dot(p, w1_ref[...], preferred_element_type=jnp.float32)
        a = jnp.maximum(a + b1_ref[0], 0.0)
        return jnp.dot(a, w2_ref[...],
                       preferred_element_type=jnp.float32) + b2_ref[0]

    gx = head(pq_ref[...])
    hx = head(pc_ref[...])
    d = gx - hx
    a = jnp.sum(jnp.maximum(d, 0.0), axis=-1) * ((NODE_DEL + EDGE_DEL) * 0.5)
    b = jnp.sum(jnp.maximum(-d, 0.0), axis=-1) * ((NODE_INS + EDGE_INS) * 0.5)
    out_ref[0] = a + b


_XSPEC = pl.BlockSpec((RBLK, H), lambda r: (r, 0))
_WSPEC = pl.BlockSpec((H, H), lambda r: (0, 0))
_BSPEC = pl.BlockSpec((1, H), lambda r: (0, 0))
_BATCHSPEC = pl.BlockSpec((1, 1, RBLK), lambda r: (r, 0, 0))
_POOLSPEC = pl.BlockSpec((B, H), lambda r: (0, 0))


def _pre_call(x, Wpre, bpre, batch3):
    return pl.pallas_call(
        _pre_body,
        grid=(NRB,),
        in_specs=[_XSPEC, _WSPEC, _BSPEC, _BATCHSPEC],
        out_specs=[_XSPEC, _POOLSPEC],
        out_shape=[jax.ShapeDtypeStruct((N, H), jnp.float32),
                   jax.ShapeDtypeStruct((B, H), jnp.float32)],
    )(x, Wpre, bpre, batch3)


def _layer_call(odd, h, W1, b1, W2, b2, batch3, xres):
    in_specs = [_XSPEC, _WSPEC, _BSPEC, _WSPEC, _BSPEC, _BATCHSPEC]
    operands = [h, W1, b1, W2, b2, batch3]
    out_specs = [_XSPEC, _POOLSPEC]
    out_shape = [jax.ShapeDtypeStruct((N, H), jnp.float32),
                 jax.ShapeDtypeStruct((B, H), jnp.float32)]
    if odd:
        in_specs.append(_XSPEC)
        operands.append(xres)
        out_specs.append(_XSPEC)
        out_shape.append(jax.ShapeDtypeStruct((N, H), jnp.float32))
    return pl.pallas_call(
        functools.partial(_layer_body, odd),
        grid=(NRB,),
        in_specs=in_specs,
        out_specs=out_specs,
        out_shape=out_shape,
    )(*operands)


def _final_call(pq, pc, Wp1, bp1, Wp2, bp2):
    full = lambda shape: pl.BlockSpec(shape, lambda: tuple(0 for _ in shape))
    return pl.pallas_call(
        _final_body,
        in_specs=[full((B, H * (L + 1))), full((B, H * (L + 1))),
                  full((H * (L + 1), H)), full((1, H)),
                  full((H, OUT)), full((1, OUT))],
        out_specs=full((1, B)),
        out_shape=jax.ShapeDtypeStruct((1, B), jnp.float32),
    )(pq, pc, Wp1, bp1, Wp2, bp2)


def _sc_h_one(x, src, dst):
    return _sc_aggregate_kernel()(x, src, dst)


def kernel(query_x, query_edge_index, query_batch, corpus_x,
           corpus_edge_index, corpus_batch, Wpre, bpre, Wc1, bc1, Wc2, bc2,
           Wp1, bp1, Wp2, bp2):
    qb3 = query_batch.reshape(NRB, 1, RBLK)
    cb3 = corpus_batch.reshape(NRB, 1, RBLK)

    pad_src = jnp.arange(EPAD, dtype=jnp.int32) % 256
    pad_dst = N + jnp.arange(EPAD, dtype=jnp.int32) % 8

    def prep_edges(ei):
        src = jnp.concatenate([ei[0], pad_src])
        dst = jnp.concatenate([ei[1], pad_dst])
        return (src.reshape(NTILE, NCHUNK, ECHUNK),
                dst.reshape(NTILE, NCHUNK, ECHUNK))

    qsrc, qdst = prep_edges(query_edge_index)
    csrc, cdst = prep_edges(corpus_edge_index)
    bpre2 = bpre.reshape(1, H)

    xq, pq0 = _pre_call(query_x, Wpre, bpre2, qb3)
    xc, pc0 = _pre_call(corpus_x, Wpre, bpre2, cb3)
    pooled_q, pooled_c = [pq0], [pc0]
    xres_q, xres_c = xq, xc
    for i in range(L):
        hq = _sc_h_one(xq, qsrc, qdst)
        hc = _sc_h_one(xc, csrc, cdst)
        b1i, b2i = bc1[i].reshape(1, H), bc2[i].reshape(1, H)
        outs_q = _layer_call(i & 1, hq, Wc1[i], b1i, Wc2[i], b2i, qb3, xres_q)
        outs_c = _layer_call(i & 1, hc, Wc1[i], b1i, Wc2[i], b2i, cb3, xres_c)
        if i & 1:
            xq, pqi, xres_q = outs_q
            xc, pci, xres_c = outs_c
        else:
            xq, pqi = outs_q
            xc, pci = outs_c
        pooled_q.append(pqi)
        pooled_c.append(pci)

    pallq = jnp.concatenate(pooled_q, axis=-1)
    pallc = jnp.concatenate(pooled_c, axis=-1)
    score = _final_call(pallq, pallc, Wp1, bp1.reshape(1, H),
                        Wp2, bp2.reshape(1, OUT))
    return score.reshape(B)
